# baseline (speedup 1.0000x reference)
import jax
import jax.numpy as jnp
from jax.experimental import pallas as pl
from jax.experimental.pallas import tpu as pltpu

_N = 20000
_D = 1024
_OUT_B = 320
_BM = 1000


def _fused_heads(x_ref, wb_ref, bb_ref, wi_ref, bi_ref, ob_ref, oi_ref):
    x = x_ref[...]
    ob_ref[...] = (
        jnp.dot(x, wb_ref[...], preferred_element_type=jnp.float32) + bb_ref[...]
    )
    oi_ref[...] = (
        jnp.dot(x, wi_ref[...], preferred_element_type=jnp.float32) + bi_ref[...]
    )


def kernel(x, W_bbox, b_bbox, W_iou, b_iou):
    if x.ndim > 2:
        x = x.reshape(x.shape[0], -1)
    n, d = x.shape
    out_b = W_bbox.shape[1]
    bb2 = b_bbox.reshape(1, out_b)
    bi2 = b_iou.reshape(1, 1)

    grid = (n // _BM,)
    deltas, iou = pl.pallas_call(
        _fused_heads,
        grid=grid,
        in_specs=[
            pl.BlockSpec((_BM, d), lambda i: (i, 0)),
            pl.BlockSpec((d, out_b), lambda i: (0, 0)),
            pl.BlockSpec((1, out_b), lambda i: (0, 0)),
            pl.BlockSpec((d, 1), lambda i: (0, 0)),
            pl.BlockSpec((1, 1), lambda i: (0, 0)),
        ],
        out_specs=[
            pl.BlockSpec((_BM, out_b), lambda i: (i, 0)),
            pl.BlockSpec((_BM, 1), lambda i: (i, 0)),
        ],
        out_shape=[
            jax.ShapeDtypeStruct((n, out_b), jnp.float32),
            jax.ShapeDtypeStruct((n, 1), jnp.float32),
        ],
        compiler_params=pltpu.CompilerParams(
            dimension_semantics=("arbitrary",),
        ),
    )(x, W_bbox, bb2, W_iou, bi2)
    return (deltas, iou)
